# baseline (speedup 1.0000x reference)
import jax
import jax.numpy as jnp
from jax.experimental import pallas as pl
from jax.experimental.pallas import tpu as pltpu

SIZE = 1024
LATENT_DIM = 256
BETA_C = 0.25
N_ROWS = 4 * 8 * 32 * 32
TILE = 1024
N_TILES = N_ROWS // TILE


def _vq_body(zr_ref, emb_ref, zq_ref, enc_ref, idx_ref, loss_ref, perp_ref,
             counts_ref, loss_acc_ref, en_ref, e2_ref, iw_ref, idx_s_ref):
    t = pl.program_id(0)

    e = emb_ref[...]

    @pl.when(t == 0)
    def _init():
        counts_ref[...] = jnp.zeros_like(counts_ref)
        loss_acc_ref[0] = 0.0
        en_ref[...] = jnp.sum(e * e, axis=1, keepdims=True)
        e2_ref[...] = -2.0 * e
        rsel = jax.lax.broadcasted_iota(jnp.int32, (2, SIZE), 0)
        col = jax.lax.broadcasted_iota(jnp.int32, (2, SIZE), 1)
        iw_ref[...] = jnp.where(rsel == 0, col, 1).astype(jnp.float32)

    zr = zr_ref[...]
    en = en_ref[...]

    ezT = jax.lax.dot_general(e2_ref[...], zr, (((1,), (1,)), ((), ())),
                              preferred_element_type=jnp.float32)
    sT = en + ezT

    minv = jnp.min(sT, axis=0)
    eq_f = (sT == minv[None, :]).astype(jnp.float32)
    res = jax.lax.dot_general(iw_ref[...], eq_f, (((1,), (0,)), ((), ())),
                              preferred_element_type=jnp.float32)
    idx_s_ref[0] = res[0].astype(jnp.int32)

    @pl.when(jnp.max(res[1]) > 1.5)
    def _tie_fallback():
        code_iota_T = jax.lax.broadcasted_iota(jnp.int32, (SIZE, TILE), 0)
        idx_s_ref[0] = jnp.min(
            jnp.where(sT == minv[None, :], code_iota_T, SIZE), axis=0)

    idx = idx_s_ref[0]

    code_iota = jax.lax.broadcasted_iota(jnp.int32, (TILE, SIZE), 1)
    oh = (code_iota == idx[:, None]).astype(jnp.float32)
    enc_ref[...] = oh
    zq = jax.lax.dot_general(oh, e, (((1,), (0,)), ((), ())),
                             preferred_element_type=jnp.float32)
    zq_ref[...] = zr + (zq - zr)

    ones_row = jnp.ones((1, TILE), jnp.float32)
    counts_ref[...] += jax.lax.dot_general(
        ones_row, oh, (((1,), (0,)), ((), ())),
        preferred_element_type=jnp.float32)
    idx_ref[0, 0] = idx

    loss_acc_ref[0] += jnp.sum(zr * zr) + jnp.sum(minv)

    @pl.when(t == N_TILES - 1)
    def _finish():
        loss_ref[0, 0] = BETA_C * loss_acc_ref[0] / (N_ROWS * LATENT_DIM)
        e_mean = counts_ref[...] / N_ROWS
        perp_ref[0, 0] = jnp.exp(-jnp.sum(e_mean * jnp.log(e_mean + 1e-10)))


def kernel(z, embedding_weight):
    zp = jnp.transpose(z, (0, 2, 3, 4, 1)).reshape(N_ROWS, LATENT_DIM)
    zq, enc, idx3, loss, perp = pl.pallas_call(
        _vq_body,
        grid=(N_TILES,),
        in_specs=[
            pl.BlockSpec((TILE, LATENT_DIM), lambda t: (t, 0)),
            pl.BlockSpec((SIZE, LATENT_DIM), lambda t: (0, 0)),
        ],
        out_specs=[
            pl.BlockSpec((TILE, LATENT_DIM), lambda t: (t, 0)),
            pl.BlockSpec((TILE, SIZE), lambda t: (t, 0)),
            pl.BlockSpec((1, 1, TILE), lambda t: (t, 0, 0)),
            pl.BlockSpec((1, 1), lambda t: (0, 0), memory_space=pltpu.SMEM),
            pl.BlockSpec((1, 1), lambda t: (0, 0), memory_space=pltpu.SMEM),
        ],
        out_shape=[
            jax.ShapeDtypeStruct((N_ROWS, LATENT_DIM), jnp.float32),
            jax.ShapeDtypeStruct((N_ROWS, SIZE), jnp.float32),
            jax.ShapeDtypeStruct((N_TILES, 1, TILE), jnp.int32),
            jax.ShapeDtypeStruct((1, 1), jnp.float32),
            jax.ShapeDtypeStruct((1, 1), jnp.float32),
        ],
        scratch_shapes=[
            pltpu.VMEM((1, SIZE), jnp.float32),
            pltpu.SMEM((1,), jnp.float32),
            pltpu.VMEM((SIZE, 1), jnp.float32),
            pltpu.VMEM((SIZE, LATENT_DIM), jnp.float32),
            pltpu.VMEM((2, SIZE), jnp.float32),
            pltpu.VMEM((1, TILE), jnp.int32),
        ],
    )(zp, embedding_weight)

    z_q_out = jnp.transpose(zq.reshape(4, 8, 32, 32, LATENT_DIM), (0, 4, 1, 2, 3))
    min_idx = idx3.reshape(N_ROWS, 1)
    return (z_q_out, loss[0, 0], perp[0, 0], enc, min_idx)


# speedup vs baseline: 1.2516x; 1.2516x over previous
import jax
import jax.numpy as jnp
from jax.experimental import pallas as pl
from jax.experimental.pallas import tpu as pltpu

SIZE = 1024
LATENT_DIM = 256
BETA_C = 0.25
N_ROWS = 4 * 8 * 32 * 32
TILE = 2048
N_TILES = N_ROWS // TILE


def _vq_body(zr_ref, emb_ref, zq_ref, enc_ref, idx_ref, loss_ref, perp_ref,
             counts_ref, loss_acc_ref, en_ref, e2_ref):
    t = pl.program_id(0)

    e = emb_ref[...]

    @pl.when(t == 0)
    def _init():
        counts_ref[...] = jnp.zeros_like(counts_ref)
        loss_acc_ref[0] = 0.0
        en_ref[...] = jnp.sum(e * e, axis=1, keepdims=True)
        e2_ref[...] = -2.0 * e

    zr = zr_ref[...]
    en = en_ref[...]

    ezT = jax.lax.dot_general(e2_ref[...], zr, (((1,), (1,)), ((), ())),
                              preferred_element_type=jnp.float32)
    sT = en + ezT

    minv = jnp.min(sT, axis=0)
    code_iota_T = jax.lax.broadcasted_iota(jnp.int32, (SIZE, TILE), 0)
    idx = jnp.min(jnp.where(sT == minv[None, :], code_iota_T, SIZE), axis=0)

    code_iota = jax.lax.broadcasted_iota(jnp.int32, (TILE, SIZE), 1)
    oh = (code_iota == idx[:, None]).astype(jnp.float32)
    enc_ref[...] = oh
    zq = jax.lax.dot_general(oh, e, (((1,), (0,)), ((), ())),
                             preferred_element_type=jnp.float32)
    zq_ref[...] = zr + (zq - zr)

    ones_row = jnp.ones((1, TILE), jnp.float32)
    counts_ref[...] += jax.lax.dot_general(
        ones_row, oh, (((1,), (0,)), ((), ())),
        preferred_element_type=jnp.float32)
    idx_ref[0, 0] = idx

    loss_acc_ref[0] += jnp.sum(zr * zr) + jnp.sum(minv)

    @pl.when(t == N_TILES - 1)
    def _finish():
        loss_ref[0, 0] = BETA_C * loss_acc_ref[0] / (N_ROWS * LATENT_DIM)
        e_mean = counts_ref[...] / N_ROWS
        perp_ref[0, 0] = jnp.exp(-jnp.sum(e_mean * jnp.log(e_mean + 1e-10)))


def kernel(z, embedding_weight):
    zp = jnp.transpose(z, (0, 2, 3, 4, 1)).reshape(N_ROWS, LATENT_DIM)
    zq, enc, idx3, loss, perp = pl.pallas_call(
        _vq_body,
        grid=(N_TILES,),
        in_specs=[
            pl.BlockSpec((TILE, LATENT_DIM), lambda t: (t, 0)),
            pl.BlockSpec((SIZE, LATENT_DIM), lambda t: (0, 0)),
        ],
        out_specs=[
            pl.BlockSpec((TILE, LATENT_DIM), lambda t: (t, 0)),
            pl.BlockSpec((TILE, SIZE), lambda t: (t, 0)),
            pl.BlockSpec((1, 1, TILE), lambda t: (t, 0, 0)),
            pl.BlockSpec((1, 1), lambda t: (0, 0), memory_space=pltpu.SMEM),
            pl.BlockSpec((1, 1), lambda t: (0, 0), memory_space=pltpu.SMEM),
        ],
        out_shape=[
            jax.ShapeDtypeStruct((N_ROWS, LATENT_DIM), jnp.float32),
            jax.ShapeDtypeStruct((N_ROWS, SIZE), jnp.float32),
            jax.ShapeDtypeStruct((N_TILES, 1, TILE), jnp.int32),
            jax.ShapeDtypeStruct((1, 1), jnp.float32),
            jax.ShapeDtypeStruct((1, 1), jnp.float32),
        ],
        scratch_shapes=[
            pltpu.VMEM((1, SIZE), jnp.float32),
            pltpu.SMEM((1,), jnp.float32),
            pltpu.VMEM((SIZE, 1), jnp.float32),
            pltpu.VMEM((SIZE, LATENT_DIM), jnp.float32),
        ],
    )(zp, embedding_weight)

    z_q_out = jnp.transpose(zq.reshape(4, 8, 32, 32, LATENT_DIM), (0, 4, 1, 2, 3))
    min_idx = idx3.reshape(N_ROWS, 1)
    return (z_q_out, loss[0, 0], perp[0, 0], enc, min_idx)


# speedup vs baseline: 1.2521x; 1.0004x over previous
import jax
import jax.numpy as jnp
from jax.experimental import pallas as pl
from jax.experimental.pallas import tpu as pltpu

SIZE = 1024
LATENT_DIM = 256
BETA_C = 0.25
N_ROWS = 4 * 8 * 32 * 32
TILE = 4096
N_TILES = N_ROWS // TILE


def _vq_body(zr_ref, emb_ref, zq_ref, enc_ref, idx_ref, loss_ref, perp_ref,
             counts_ref, loss_acc_ref, en_ref, e2_ref):
    t = pl.program_id(0)

    e = emb_ref[...]

    @pl.when(t == 0)
    def _init():
        counts_ref[...] = jnp.zeros_like(counts_ref)
        loss_acc_ref[0] = 0.0
        en_ref[...] = jnp.sum(e * e, axis=1, keepdims=True)
        e2_ref[...] = -2.0 * e

    zr = zr_ref[...]
    en = en_ref[...]

    ezT = jax.lax.dot_general(e2_ref[...], zr, (((1,), (1,)), ((), ())),
                              preferred_element_type=jnp.float32)
    sT = en + ezT

    minv = jnp.min(sT, axis=0)
    code_iota_T = jax.lax.broadcasted_iota(jnp.int32, (SIZE, TILE), 0)
    idx = jnp.min(jnp.where(sT == minv[None, :], code_iota_T, SIZE), axis=0)

    code_iota = jax.lax.broadcasted_iota(jnp.int32, (TILE, SIZE), 1)
    oh = (code_iota == idx[:, None]).astype(jnp.float32)
    enc_ref[...] = oh
    zq = jax.lax.dot_general(oh, e, (((1,), (0,)), ((), ())),
                             preferred_element_type=jnp.float32)
    zq_ref[...] = zr + (zq - zr)

    ones_row = jnp.ones((1, TILE), jnp.float32)
    counts_ref[...] += jax.lax.dot_general(
        ones_row, oh, (((1,), (0,)), ((), ())),
        preferred_element_type=jnp.float32)
    idx_ref[0, 0] = idx

    loss_acc_ref[0] += jnp.sum(zr * zr) + jnp.sum(minv)

    @pl.when(t == N_TILES - 1)
    def _finish():
        loss_ref[0, 0] = BETA_C * loss_acc_ref[0] / (N_ROWS * LATENT_DIM)
        e_mean = counts_ref[...] / N_ROWS
        perp_ref[0, 0] = jnp.exp(-jnp.sum(e_mean * jnp.log(e_mean + 1e-10)))


def kernel(z, embedding_weight):
    zp = jnp.transpose(z, (0, 2, 3, 4, 1)).reshape(N_ROWS, LATENT_DIM)
    zq, enc, idx3, loss, perp = pl.pallas_call(
        _vq_body,
        grid=(N_TILES,),
        in_specs=[
            pl.BlockSpec((TILE, LATENT_DIM), lambda t: (t, 0)),
            pl.BlockSpec((SIZE, LATENT_DIM), lambda t: (0, 0)),
        ],
        out_specs=[
            pl.BlockSpec((TILE, LATENT_DIM), lambda t: (t, 0)),
            pl.BlockSpec((TILE, SIZE), lambda t: (t, 0)),
            pl.BlockSpec((1, 1, TILE), lambda t: (t, 0, 0)),
            pl.BlockSpec((1, 1), lambda t: (0, 0), memory_space=pltpu.SMEM),
            pl.BlockSpec((1, 1), lambda t: (0, 0), memory_space=pltpu.SMEM),
        ],
        out_shape=[
            jax.ShapeDtypeStruct((N_ROWS, LATENT_DIM), jnp.float32),
            jax.ShapeDtypeStruct((N_ROWS, SIZE), jnp.float32),
            jax.ShapeDtypeStruct((N_TILES, 1, TILE), jnp.int32),
            jax.ShapeDtypeStruct((1, 1), jnp.float32),
            jax.ShapeDtypeStruct((1, 1), jnp.float32),
        ],
        scratch_shapes=[
            pltpu.VMEM((1, SIZE), jnp.float32),
            pltpu.SMEM((1,), jnp.float32),
            pltpu.VMEM((SIZE, 1), jnp.float32),
            pltpu.VMEM((SIZE, LATENT_DIM), jnp.float32),
        ],
    )(zp, embedding_weight)

    z_q_out = jnp.transpose(zq.reshape(4, 8, 32, 32, LATENT_DIM), (0, 4, 1, 2, 3))
    min_idx = idx3.reshape(N_ROWS, 1)
    return (z_q_out, loss[0, 0], perp[0, 0], enc, min_idx)
